# baseline (speedup 1.0000x reference)
import functools
import jax
import jax.numpy as jnp
from jax import lax
from jax.experimental import pallas as pl
from jax.experimental.pallas import tpu as pltpu
from jax.experimental.pallas import tpu_sc as plsc

_L = 16


def kernel(date, table):
    n, c = date.shape
    e = table.shape[1]
    b = n * c
    info = plsc.get_sparse_core_info()
    nw = info.num_cores * info.num_subcores
    b_per_w = b // nw
    chunk = 2048
    n_chunks = b_per_w // chunk

    idx_flat = date.reshape(b)
    table_flat = table.reshape(table.shape[0] * e)

    mesh = plsc.VectorSubcoreMesh(core_axis_name="c", subcore_axis_name="s")

    @functools.partial(
        pl.kernel,
        mesh=mesh,
        out_type=jax.ShapeDtypeStruct((b * e,), jnp.float32),
        scratch_types=[
            pltpu.VMEM((table.shape[0] * e,), jnp.float32),
            pltpu.VMEM((chunk,), jnp.int32),
            pltpu.VMEM((chunk * e,), jnp.float32),
        ],
        compiler_params=pltpu.CompilerParams(needs_layout_passes=False),
    )
    def _k(idx_hbm, tab_hbm, out_hbm, tab_v, idx_v, rows_v):
        wid = lax.axis_index("s") * info.num_cores + lax.axis_index("c")
        base = wid * b_per_w
        pltpu.sync_copy(tab_hbm, tab_v)
        iota = lax.iota(jnp.int32, _L)

        def chunk_body(ci, _):
            off = base + ci * chunk
            pltpu.sync_copy(idx_hbm.at[pl.ds(off, chunk)], idx_v)

            def step(i, _):
                dvec = idx_v[pl.ds(i * _L, _L)]
                rowbase = dvec * e
                posbase = iota * e + i * (e * _L)
                for k in range(e):
                    vals = plsc.load_gather(tab_v, [rowbase + k])
                    plsc.store_scatter(rows_v, [posbase + k], vals)
                return 0

            lax.fori_loop(0, chunk // _L, step, 0, unroll=False)
            pltpu.sync_copy(rows_v, out_hbm.at[pl.ds(off * e, chunk * e)])
            return 0

        lax.fori_loop(0, n_chunks, chunk_body, 0, unroll=False)

    out = _k(idx_flat, table_flat)
    return out.reshape(n, c, e)


# speedup vs baseline: 26.4165x; 26.4165x over previous
import jax
import jax.numpy as jnp
from jax.experimental import pallas as pl


def _embed_kernel(dt_ref, table_ref, out_ref):
    d3 = dt_ref[...][:, None, :]
    t = table_ref[...]
    t0 = t[0][:, None]
    t1 = t[1][:, None]
    t2 = t[2][:, None]
    t3 = t[3][:, None]
    out_ref[...] = jnp.where(
        d3 < 2,
        jnp.where(d3 == 0, t0, t1),
        jnp.where(d3 == 2, t2, t3),
    )


def kernel(date, table):
    n, c = date.shape
    e = table.shape[1]
    dt = jnp.swapaxes(date, 0, 1)
    block = 512
    grid = (n // block,)
    out_t = pl.pallas_call(
        _embed_kernel,
        grid=grid,
        in_specs=[
            pl.BlockSpec((c, block), lambda i: (0, i)),
            pl.BlockSpec((4, e), lambda i: (0, 0)),
        ],
        out_specs=pl.BlockSpec((c, e, block), lambda i: (0, 0, i)),
        out_shape=jax.ShapeDtypeStruct((c, e, n), table.dtype),
    )(dt, table)
    return jnp.transpose(out_t, (2, 0, 1))
